# baseline (speedup 1.0000x reference)
import functools

import jax
import jax.numpy as jnp
from jax import lax
from jax.experimental import pallas as pl
from jax.experimental.pallas import tpu as pltpu
from jax.experimental.pallas import tpu_sc as plsc

N = 10000
E = 160000
EP = 80000
RB = 1000
NRB = N // RB


def _mm1_body(x_ref, w_ref, aa_ref, h_ref, ee_ref):
    c = pl.program_id(1)
    hb = jnp.dot(x_ref[...], w_ref[...], preferred_element_type=jnp.float32)
    h_ref[0] = hb

    @pl.when(c == 0)
    def _():
        ee_ref[...] = jnp.zeros_like(ee_ref)

    ee_ref[...] += jnp.dot(hb, aa_ref[...], preferred_element_type=jnp.float32)


def _mm1(x, Wg1, AA1, C, K):
    return pl.pallas_call(
        _mm1_body,
        grid=(NRB, C),
        in_specs=[
            pl.BlockSpec((RB, K), lambda r, c: (r, 0)),
            pl.BlockSpec((K, 128), lambda r, c: (0, c)),
            pl.BlockSpec((128, 8), lambda r, c: (c, 0)),
        ],
        out_specs=[
            pl.BlockSpec((1, RB, 128), lambda r, c: (c, r, 0)),
            pl.BlockSpec((RB, 8), lambda r, c: (r, 0)),
        ],
        out_shape=[
            jax.ShapeDtypeStruct((C, N, 128), jnp.float32),
            jax.ShapeDtypeStruct((N, 8), jnp.float32),
        ],
    )(x, Wg1, AA1)


def _mid_body(out1_ref, den_ref, wg2_ref, aa2_ref, h2t_ref, ee2_ref):
    den = jnp.maximum(den_ref[0] + den_ref[1], 1e-9)
    acc = jnp.zeros((RB, 256), jnp.float32)
    for c in range(8):
        xc = out1_ref[c] / den[:, c // 2][:, None]
        xc = jnp.where(xc > 0, xc, jnp.exp(jnp.minimum(xc, 0.0)) - 1.0)
        acc += jnp.dot(xc, wg2_ref[c], preferred_element_type=jnp.float32)
    h2t_ref[0] = acc[:, :128]
    h2t_ref[1] = acc[:, 128:]
    ee2_ref[...] = jnp.dot(acc, aa2_ref[...], preferred_element_type=jnp.float32)


def _mid(out1, den2x, Wg2r, AA2):
    return pl.pallas_call(
        _mid_body,
        grid=(NRB,),
        in_specs=[
            pl.BlockSpec((8, RB, 128), lambda r: (0, r, 0)),
            pl.BlockSpec((2, RB, 4), lambda r: (0, r, 0)),
            pl.BlockSpec((8, 128, 256), lambda r: (0, 0, 0)),
            pl.BlockSpec((256, 8), lambda r: (0, 0)),
        ],
        out_specs=[
            pl.BlockSpec((2, RB, 128), lambda r: (0, r, 0)),
            pl.BlockSpec((RB, 8), lambda r: (r, 0)),
        ],
        out_shape=[
            jax.ShapeDtypeStruct((2, N, 128), jnp.float32),
            jax.ShapeDtypeStruct((N, 8), jnp.float32),
        ],
    )(out1, den2x, Wg2r, AA2)


def _uv_body(out2_ref, den_ref, wm1_ref, b1_ref, uv_ref):
    den = jnp.maximum(den_ref[0] + den_ref[1], 1e-9)
    xs = []
    for c in range(2):
        div = jnp.concatenate(
            [jnp.broadcast_to(den[:, 2 * c][:, None], (RB, 64)),
             jnp.broadcast_to(den[:, 2 * c + 1][:, None], (RB, 64))], axis=1)
        xs.append(out2_ref[c] / div)
    u = (jnp.dot(xs[0], wm1_ref[0:128], preferred_element_type=jnp.float32)
         + jnp.dot(xs[1], wm1_ref[128:256], preferred_element_type=jnp.float32)
         + b1_ref[...])
    v = (jnp.dot(xs[0], wm1_ref[256:384], preferred_element_type=jnp.float32)
         + jnp.dot(xs[1], wm1_ref[384:512], preferred_element_type=jnp.float32))
    uv_ref[0] = u
    uv_ref[1] = v


def _uv(out2, den2x, Wm1, b1):
    return pl.pallas_call(
        _uv_body,
        grid=(NRB,),
        in_specs=[
            pl.BlockSpec((2, RB, 128), lambda r: (0, r, 0)),
            pl.BlockSpec((2, RB, 4), lambda r: (0, r, 0)),
            pl.BlockSpec((512, 256), lambda r: (0, 0)),
            pl.BlockSpec((1, 256), lambda r: (0, 0)),
        ],
        out_specs=pl.BlockSpec((2, RB, 256), lambda r: (0, r, 0)),
        out_shape=jax.ShapeDtypeStruct((2, N, 256), jnp.float32),
    )(out2, den2x, Wm1, b1)


SC_NC = 2
SC_NS = 16
NWK = SC_NC * SC_NS
EW = E // NWK
WIN = 128
NWIN = (EW + WIN - 1) // WIN
EWPAD = NWIN * WIN
DEN_PAD = 40960
DEN_TILE = DEN_PAD // SC_NS

_SC_MESH = dict(core_axis_name="c", subcore_axis_name="s")


def _alpha_body(ee_ref, src_ref, dst_ref, alphaT_ref, den_ref,
                eebuf, sbuf, dbuf, ab0, ab1, ab2, ab3,
                scw0, scw1, scw2, scw3, zwin, den_sp):
    c = lax.axis_index("c")
    s = lax.axis_index("s")
    wid = s * SC_NC + c
    base = wid * EW
    scws = (scw0, scw1, scw2, scw3)
    abufs = (ab0, ab1, ab2, ab3)
    pltpu.sync_copy(ee_ref, eebuf)
    pltpu.sync_copy(src_ref.at[pl.ds(base, EW)], sbuf.at[pl.ds(0, EW)])
    pltpu.sync_copy(dst_ref.at[pl.ds(base, EW)], dbuf.at[pl.ds(0, EW)])

    def zloop(i, _):
        zwin[pl.ds(i * 16, 16)] = jnp.zeros((16,), jnp.float32)
        return 0
    lax.fori_loop(0, DEN_TILE // 16, zloop, 0)
    pltpu.sync_copy(zwin, den_sp.at[pl.ds(s * DEN_TILE, DEN_TILE)])
    plsc.subcore_barrier()
    iota = lax.iota(jnp.int32, 16)

    def window(w, _):
        def step(j, _):
            off = w * WIN + j * 16
            eidx = off + iota
            valid = eidx < EW
            s16 = jnp.clip(sbuf[pl.ds(off, 16)], 0, N - 1)
            d16 = jnp.clip(dbuf[pl.ds(off, 16)], 0, N - 1)
            d4 = d16 * 4
            dump = 4 * N + wid * 16 + iota
            for h in range(4):
                el = plsc.load_gather(eebuf, [s16 * 8 + h])
                er = plsc.load_gather(eebuf, [d16 * 8 + 4 + h])
                e = el + er
                e = jnp.where(e >= 0, e, 0.2 * e)
                abufs[h][pl.ds(off, 16)] = jnp.exp(e)
                scws[h][pl.ds(j * 16, 16)] = jnp.where(valid, d4 + h, dump)
            return 0
        lax.fori_loop(0, 8, step, 0)
        for h in range(4):
            pltpu.sync_copy(abufs[h].at[pl.ds(w * WIN, WIN)],
                            den_sp.at[scws[h]], add=True)
        return 0
    lax.fori_loop(0, NWIN, window, 0)
    for h in range(4):
        pltpu.sync_copy(abufs[h].at[pl.ds(0, EW)],
                        alphaT_ref.at[pl.ds(h * E + base, EW)])
    plsc.subcore_barrier()
    pltpu.sync_copy(den_sp.at[pl.ds(s * DEN_TILE, DEN_TILE)],
                    den_ref.at[c, pl.ds(s * DEN_TILE, DEN_TILE)])


def _alpha_sc(ee_flat, src, dst):
    f = pl.kernel(
        _alpha_body,
        out_type=[
            jax.ShapeDtypeStruct((4 * E,), jnp.float32),
            jax.ShapeDtypeStruct((SC_NC, DEN_PAD), jnp.float32),
        ],
        mesh=plsc.VectorSubcoreMesh(**_SC_MESH),
        compiler_params=pltpu.CompilerParams(needs_layout_passes=False),
        scratch_types=[
            pltpu.VMEM((8 * N,), jnp.float32),
            pltpu.VMEM((EWPAD,), jnp.int32),
            pltpu.VMEM((EWPAD,), jnp.int32),
            pltpu.VMEM((EWPAD,), jnp.float32),
            pltpu.VMEM((EWPAD,), jnp.float32),
            pltpu.VMEM((EWPAD,), jnp.float32),
            pltpu.VMEM((EWPAD,), jnp.float32),
            pltpu.VMEM((WIN,), jnp.int32),
            pltpu.VMEM((WIN,), jnp.int32),
            pltpu.VMEM((WIN,), jnp.int32),
            pltpu.VMEM((WIN,), jnp.int32),
            pltpu.VMEM((DEN_TILE,), jnp.float32),
            pltpu.VMEM_SHARED((DEN_PAD,), jnp.float32),
        ],
    )
    return f(ee_flat, src, dst)


MWIN = 96
MNWIN = 54
MEPAD = MNWIN * MWIN


def _mlp_body(uvt_ref, src_ref, dst_ref, wm2_ref, b2_ref, out_ref,
              sbuf, dbuf, scob, bu0, bv0, bu1, bv1, wmb, b2b, sm0, sm1):
    c = lax.axis_index("c")
    s = lax.axis_index("s")
    wid = s * SC_NC + c
    base = wid * EW
    pltpu.sync_copy(src_ref.at[pl.ds(base, EW)], sbuf.at[pl.ds(0, EW)])
    pltpu.sync_copy(dst_ref.at[pl.ds(base, EW)], dbuf.at[pl.ds(0, EW)])
    pltpu.sync_copy(wm2_ref, wmb)
    pltpu.sync_copy(b2_ref, b2b)

    def san(i, _):
        sbuf[pl.ds(i * 16, 16)] = jnp.clip(sbuf[pl.ds(i * 16, 16)], 0, N - 1)
        dbuf[pl.ds(i * 16, 16)] = jnp.clip(dbuf[pl.ds(i * 16, 16)], 0, N - 1) + N
        return 0
    lax.fori_loop(0, MEPAD // 16, san, 0)
    b2s = b2b[pl.ds(0, 16)][0]
    iota = lax.iota(jnp.int32, 16)

    def start(w, bu, bv, sm):
        pltpu.async_copy(uvt_ref.at[sbuf.at[pl.ds(w * MWIN, MWIN)]], bu, sm)
        pltpu.async_copy(uvt_ref.at[dbuf.at[pl.ds(w * MWIN, MWIN)]], bv, sm)

    def wait(w, bu, bv, sm):
        pltpu.make_async_copy(uvt_ref.at[sbuf.at[pl.ds(w * MWIN, MWIN)]], bu, sm).wait()
        pltpu.make_async_copy(uvt_ref.at[dbuf.at[pl.ds(w * MWIN, MWIN)]], bv, sm).wait()

    def compute(w, bu, bv):
        def group(g, _):
            e16 = g * 16 + iota
            scv = jnp.zeros((16,), jnp.float32) + b2s
            for fc in range(16):
                w2v = wmb[pl.ds(fc * 16, 16)]
                for k in range(16):
                    f = fc * 16 + k
                    f16 = jnp.full((16,), f, jnp.int32)
                    t = (plsc.load_gather(bu, [e16, f16])
                         + plsc.load_gather(bv, [e16, f16]))
                    scv = scv + jnp.maximum(t, 0.0) * w2v[k]
            scob[pl.ds(w * MWIN + g * 16, 16)] = scv
            return 0
        lax.fori_loop(0, MWIN // 16, group, 0)

    start(0, bu0, bv0, sm0)

    def pair(p, _):
        w0 = p * 2
        w1 = w0 + 1
        start(w1, bu1, bv1, sm1)
        wait(w0, bu0, bv0, sm0)
        compute(w0, bu0, bv0)

        @pl.when(p < MNWIN // 2 - 1)
        def _():
            start(w0 + 2, bu0, bv0, sm0)
        wait(w1, bu1, bv1, sm1)
        compute(w1, bu1, bv1)
        return 0
    lax.fori_loop(0, MNWIN // 2, pair, 0)
    pltpu.sync_copy(scob.at[pl.ds(0, EW)], out_ref.at[pl.ds(base, EW)])


def _mlp_sc(uvt, srccat, dstcat, wm2, b2):
    f = pl.kernel(
        _mlp_body,
        out_type=jax.ShapeDtypeStruct((E,), jnp.float32),
        mesh=plsc.VectorSubcoreMesh(**_SC_MESH),
        compiler_params=pltpu.CompilerParams(needs_layout_passes=False),
        scratch_types=[
            pltpu.VMEM((MEPAD,), jnp.int32),
            pltpu.VMEM((MEPAD,), jnp.int32),
            pltpu.VMEM((MEPAD,), jnp.float32),
            pltpu.VMEM((MWIN, 256), jnp.float32),
            pltpu.VMEM((MWIN, 256), jnp.float32),
            pltpu.VMEM((MWIN, 256), jnp.float32),
            pltpu.VMEM((MWIN, 256), jnp.float32),
            pltpu.VMEM((256,), jnp.float32),
            pltpu.VMEM((8,), jnp.float32),
            pltpu.SemaphoreType.DMA,
            pltpu.SemaphoreType.DMA,
        ],
    )
    return f(uvt, srccat, dstcat, wm2, b2)


ET = E // SC_NS
APASS = 5
PE = ET // APASS
PNWIN = 16
PEPAD = PNWIN * WIN
NPADR = 10752
ZROWS = 128
FL_A = 632
FL_B = N - 15 * FL_A


def _agg_body_gen(C, D):
    CPC = C // SC_NC

    def body(ht_ref, alphaT_ref, src_ref, dst_ref, outf_ref,
             sbuf, dbuf, alA, alB, gw0, gw1, sc0, sc1,
             rows0, rows1, sm0, sm1, ss0, ss1, acc_sp):
        core = lax.axis_index("c")
        s = lax.axis_index("s")
        ebase = s * ET
        iota = lax.iota(jnp.int32, 16)
        zrow0 = s * (NPADR // SC_NS)
        gws = (gw0, gw1)
        scs = (sc0, sc1)
        rws = (rows0, rows1)
        sms = (sm0, sm1)

        def build_idx(w, c, gw, sc):
            def step(j, _):
                off = w * WIN + j * 16
                eidx = off + iota
                valid = eidx < PE
                s16 = jnp.clip(sbuf[pl.ds(off, 16)], 0, N - 1) + c * N
                d16 = jnp.clip(dbuf[pl.ds(off, 16)], 0, N - 1)
                gw[pl.ds(j * 16, 16)] = s16
                sc[pl.ds(j * 16, 16)] = jnp.where(valid, d16, N + s * 16 + iota)
                return 0
            lax.fori_loop(0, 8, step, 0)

        def scale(w, rows):
            def group(g, _):
                a16 = alA[pl.ds(w * WIN + g * 16, 16)]
                b16 = alB[pl.ds(w * WIN + g * 16, 16)]
                for k in range(16):
                    e = g * 16 + k
                    av = a16[k]
                    bv = b16[k]
                    for v in range(4):
                        rows[e, pl.ds(v * 16, 16)] *= av
                    for v in range(4, 8):
                        rows[e, pl.ds(v * 16, 16)] *= bv
                return 0
            lax.fori_loop(0, WIN // 16, group, 0)

        def chunk_body(cc, _):
            c = cc * SC_NC + core
            ha = (c * 128) // D
            hb = (c * 128 + 64) // D

            def zfill(r, _):
                for v in range(8):
                    rows0[r, pl.ds(v * 16, 16)] = jnp.zeros((16,), jnp.float32)
                return 0
            lax.fori_loop(0, ZROWS, zfill, 0)
            for z in range(5):
                pltpu.sync_copy(rows0, acc_sp.at[pl.ds(zrow0 + z * ZROWS, ZROWS)])
            pltpu.sync_copy(rows0.at[pl.ds(0, 32)],
                            acc_sp.at[pl.ds(zrow0 + 5 * ZROWS, 32)])
            plsc.subcore_barrier()

            def pass_body(hp, _):
                eoff = ebase + hp * PE
                pltpu.sync_copy(src_ref.at[pl.ds(eoff, PE)], sbuf.at[pl.ds(0, PE)])
                pltpu.sync_copy(dst_ref.at[pl.ds(eoff, PE)], dbuf.at[pl.ds(0, PE)])
                pltpu.sync_copy(alphaT_ref.at[pl.ds(ha * E + eoff, PE)],
                                alA.at[pl.ds(0, PE)])
                pltpu.sync_copy(alphaT_ref.at[pl.ds(hb * E + eoff, PE)],
                                alB.at[pl.ds(0, PE)])

                build_idx(0, c, gw0, sc0)
                pltpu.async_copy(ht_ref.at[gw0], rows0, sm0)

                def pair(p, _):
                    w0 = p * 2
                    w1 = w0 + 1

                    @pl.when(p > 0)
                    def _():
                        pltpu.make_async_copy(rows1, acc_sp.at[sc1], ss1).wait()
                    build_idx(w1, c, gw1, sc1)
                    pltpu.async_copy(ht_ref.at[gw1], rows1, sm1)
                    pltpu.make_async_copy(ht_ref.at[gw0], rows0, sm0).wait()
                    scale(w0, rows0)
                    pltpu.async_copy(rows0, acc_sp.at[sc0], ss0, add=True)
                    pltpu.make_async_copy(ht_ref.at[gw1], rows1, sm1).wait()
                    scale(w1, rows1)

                    @pl.when(p < PNWIN // 2 - 1)
                    def _():
                        pltpu.make_async_copy(rows0, acc_sp.at[sc0], ss0).wait()
                        build_idx(w0 + 2, c, gw0, sc0)
                        pltpu.async_copy(ht_ref.at[gw0], rows0, sm0)
                    pltpu.async_copy(rows1, acc_sp.at[sc1], ss1, add=True)
                    return 0
                lax.fori_loop(0, PNWIN // 2, pair, 0)
                pltpu.make_async_copy(rows0, acc_sp.at[sc0], ss0).wait()
                pltpu.make_async_copy(rows1, acc_sp.at[sc1], ss1).wait()
                return 0
            lax.fori_loop(0, APASS, pass_body, 0)
            plsc.subcore_barrier()

            @pl.when(s < SC_NS - 1)
            def _():
                pltpu.sync_copy(acc_sp.at[pl.ds(s * FL_A, FL_A)],
                                outf_ref.at[pl.ds(c * N + s * FL_A, FL_A)])

            @pl.when(s == SC_NS - 1)
            def _():
                pltpu.sync_copy(acc_sp.at[pl.ds(15 * FL_A, FL_B)],
                                outf_ref.at[pl.ds(c * N + 15 * FL_A, FL_B)])
            plsc.subcore_barrier()
            return 0
        lax.fori_loop(0, CPC, chunk_body, 0)
    return body


def _agg_sc(ht_flat, alphaT, src, dst, C, D):
    f = pl.kernel(
        _agg_body_gen(C, D),
        out_type=jax.ShapeDtypeStruct((C * N, 128), jnp.float32),
        mesh=plsc.VectorSubcoreMesh(**_SC_MESH),
        compiler_params=pltpu.CompilerParams(needs_layout_passes=False),
        scratch_types=[
            pltpu.VMEM((PEPAD,), jnp.int32),
            pltpu.VMEM((PEPAD,), jnp.int32),
            pltpu.VMEM((PEPAD,), jnp.float32),
            pltpu.VMEM((PEPAD,), jnp.float32),
            pltpu.VMEM((WIN,), jnp.int32),
            pltpu.VMEM((WIN,), jnp.int32),
            pltpu.VMEM((WIN,), jnp.int32),
            pltpu.VMEM((WIN,), jnp.int32),
            pltpu.VMEM((WIN, 128), jnp.float32),
            pltpu.VMEM((WIN, 128), jnp.float32),
            pltpu.SemaphoreType.DMA,
            pltpu.SemaphoreType.DMA,
            pltpu.SemaphoreType.DMA,
            pltpu.SemaphoreType.DMA,
            pltpu.VMEM_SHARED((NPADR, 128), jnp.float32),
        ],
    )
    return f(ht_flat, alphaT, src, dst)


def _edge_phase(ht, ee, src, dst, C, D):
    alphaT, den2x = _alpha_sc(ee.reshape(8 * N), src, dst)
    outf = _agg_sc(ht.reshape(C * N, 128), alphaT, src, dst, C, D)
    return outf.reshape(C, N, 128), den2x[:, :4 * N].reshape(2, N, 4)


def kernel(x, block_edge_index, pos_edge_index, neg_edge_index,
           Wg1, al1, ar1, Wg2, al2, ar2, Wm1, bm1, Wm2, bm2):
    eye4 = jnp.eye(4, dtype=jnp.float32)
    AA1 = jnp.concatenate(
        [(eye4[:, None, :] * al1[:, :, None]).reshape(1024, 4),
         (eye4[:, None, :] * ar1[:, :, None]).reshape(1024, 4)], axis=1)
    AA2 = jnp.concatenate(
        [(eye4[:, None, :] * al2[:, :, None]).reshape(256, 4),
         (eye4[:, None, :] * ar2[:, :, None]).reshape(256, 4)], axis=1)
    Wg2r = Wg2.reshape(8, 128, 256)
    src, dst = block_edge_index[0], block_edge_index[1]

    h1t, ee1 = _mm1(x, Wg1, AA1, 8, 512)
    out1, den1 = _edge_phase(h1t, ee1, src, dst, 8, 256)
    h2t, ee2 = _mid(out1, den1, Wg2r, AA2)
    out2, den2 = _edge_phase(h2t, ee2, src, dst, 2, 64)
    uv = _uv(out2, den2, Wm1, bm1.reshape(1, 256))
    srccat = jnp.concatenate([pos_edge_index[0], neg_edge_index[0]])
    dstcat = jnp.concatenate([pos_edge_index[1], neg_edge_index[1]])
    scores = _mlp_sc(uv.reshape(2 * N, 256), srccat, dstcat,
                     Wm2.reshape(256), jnp.broadcast_to(bm2, (8,)))
    return (scores[:EP], scores[EP:])


# speedup vs baseline: 1.0748x; 1.0748x over previous
import functools

import jax
import jax.numpy as jnp
from jax import lax
from jax.experimental import pallas as pl
from jax.experimental.pallas import tpu as pltpu
from jax.experimental.pallas import tpu_sc as plsc

N = 10000
E = 160000
EP = 80000
RB = 1000
NRB = N // RB


def _mm1_body(x_ref, w_ref, aa_ref, h_ref, ee_ref):
    c = pl.program_id(1)
    hb = jnp.dot(x_ref[...], w_ref[...], preferred_element_type=jnp.float32)
    h_ref[0] = hb.astype(jnp.bfloat16)

    @pl.when(c == 0)
    def _():
        ee_ref[...] = jnp.zeros_like(ee_ref)

    ee_ref[...] += jnp.dot(hb, aa_ref[...], preferred_element_type=jnp.float32)


def _mm1(x, Wg1, AA1, C, K):
    return pl.pallas_call(
        _mm1_body,
        grid=(NRB, C),
        in_specs=[
            pl.BlockSpec((RB, K), lambda r, c: (r, 0)),
            pl.BlockSpec((K, 128), lambda r, c: (0, c)),
            pl.BlockSpec((128, 8), lambda r, c: (c, 0)),
        ],
        out_specs=[
            pl.BlockSpec((1, RB, 128), lambda r, c: (c, r, 0)),
            pl.BlockSpec((RB, 8), lambda r, c: (r, 0)),
        ],
        out_shape=[
            jax.ShapeDtypeStruct((C, N, 128), jnp.bfloat16),
            jax.ShapeDtypeStruct((N, 8), jnp.float32),
        ],
    )(x, Wg1, AA1)


def _mid_body(out1_ref, den_ref, wg2_ref, aa2_ref, h2t_ref, ee2_ref):
    den = jnp.maximum(den_ref[0] + den_ref[1], 1e-9)
    acc = jnp.zeros((RB, 256), jnp.float32)
    for c in range(8):
        xc = out1_ref[c] / den[:, c // 2][:, None]
        xc = jnp.where(xc > 0, xc, jnp.exp(jnp.minimum(xc, 0.0)) - 1.0)
        acc += jnp.dot(xc, wg2_ref[c], preferred_element_type=jnp.float32)
    h2t_ref[0] = acc[:, :128].astype(jnp.bfloat16)
    h2t_ref[1] = acc[:, 128:].astype(jnp.bfloat16)
    ee2_ref[...] = jnp.dot(acc, aa2_ref[...], preferred_element_type=jnp.float32)


def _mid(out1, den2x, Wg2r, AA2):
    return pl.pallas_call(
        _mid_body,
        grid=(NRB,),
        in_specs=[
            pl.BlockSpec((8, RB, 128), lambda r: (0, r, 0)),
            pl.BlockSpec((2, RB, 4), lambda r: (0, r, 0)),
            pl.BlockSpec((8, 128, 256), lambda r: (0, 0, 0)),
            pl.BlockSpec((256, 8), lambda r: (0, 0)),
        ],
        out_specs=[
            pl.BlockSpec((2, RB, 128), lambda r: (0, r, 0)),
            pl.BlockSpec((RB, 8), lambda r: (r, 0)),
        ],
        out_shape=[
            jax.ShapeDtypeStruct((2, N, 128), jnp.bfloat16),
            jax.ShapeDtypeStruct((N, 8), jnp.float32),
        ],
    )(out1, den2x, Wg2r, AA2)


def _uv_body(out2_ref, den_ref, wm1_ref, b1_ref, uv_ref):
    den = jnp.maximum(den_ref[0] + den_ref[1], 1e-9)
    xs = []
    for c in range(2):
        div = jnp.concatenate(
            [jnp.broadcast_to(den[:, 2 * c][:, None], (RB, 64)),
             jnp.broadcast_to(den[:, 2 * c + 1][:, None], (RB, 64))], axis=1)
        xs.append(out2_ref[c] / div)
    u = (jnp.dot(xs[0], wm1_ref[0:128], preferred_element_type=jnp.float32)
         + jnp.dot(xs[1], wm1_ref[128:256], preferred_element_type=jnp.float32)
         + b1_ref[...])
    v = (jnp.dot(xs[0], wm1_ref[256:384], preferred_element_type=jnp.float32)
         + jnp.dot(xs[1], wm1_ref[384:512], preferred_element_type=jnp.float32))
    uv_ref[0] = u
    uv_ref[1] = v


def _uv(out2, den2x, Wm1, b1):
    return pl.pallas_call(
        _uv_body,
        grid=(NRB,),
        in_specs=[
            pl.BlockSpec((2, RB, 128), lambda r: (0, r, 0)),
            pl.BlockSpec((2, RB, 4), lambda r: (0, r, 0)),
            pl.BlockSpec((512, 256), lambda r: (0, 0)),
            pl.BlockSpec((1, 256), lambda r: (0, 0)),
        ],
        out_specs=pl.BlockSpec((2, RB, 256), lambda r: (0, r, 0)),
        out_shape=jax.ShapeDtypeStruct((2, N, 256), jnp.float32),
    )(out2, den2x, Wm1, b1)


SC_NC = 2
SC_NS = 16
NWK = SC_NC * SC_NS
EW = E // NWK
WIN = 128
NWIN = (EW + WIN - 1) // WIN
EWPAD = NWIN * WIN
DEN_PAD = 40960
DEN_TILE = DEN_PAD // SC_NS

_SC_MESH = dict(core_axis_name="c", subcore_axis_name="s")


def _alpha_body(ee_ref, src_ref, dst_ref, alphaT_ref, den_ref,
                eebuf, sbuf, dbuf, ab0, ab1, ab2, ab3,
                scw0, scw1, scw2, scw3, zwin, den_sp):
    c = lax.axis_index("c")
    s = lax.axis_index("s")
    wid = s * SC_NC + c
    base = wid * EW
    scws = (scw0, scw1, scw2, scw3)
    abufs = (ab0, ab1, ab2, ab3)
    pltpu.sync_copy(ee_ref, eebuf)
    pltpu.sync_copy(src_ref.at[pl.ds(base, EW)], sbuf.at[pl.ds(0, EW)])
    pltpu.sync_copy(dst_ref.at[pl.ds(base, EW)], dbuf.at[pl.ds(0, EW)])

    def zloop(i, _):
        zwin[pl.ds(i * 16, 16)] = jnp.zeros((16,), jnp.float32)
        return 0
    lax.fori_loop(0, DEN_TILE // 16, zloop, 0)
    pltpu.sync_copy(zwin, den_sp.at[pl.ds(s * DEN_TILE, DEN_TILE)])
    plsc.subcore_barrier()
    iota = lax.iota(jnp.int32, 16)

    def window(w, _):
        def step(j, _):
            off = w * WIN + j * 16
            eidx = off + iota
            valid = eidx < EW
            s16 = jnp.clip(sbuf[pl.ds(off, 16)], 0, N - 1)
            d16 = jnp.clip(dbuf[pl.ds(off, 16)], 0, N - 1)
            d4 = d16 * 4
            dump = 4 * N + wid * 16 + iota
            for h in range(4):
                el = plsc.load_gather(eebuf, [s16 * 8 + h])
                er = plsc.load_gather(eebuf, [d16 * 8 + 4 + h])
                e = el + er
                e = jnp.where(e >= 0, e, 0.2 * e)
                abufs[h][pl.ds(off, 16)] = jnp.exp(e)
                scws[h][pl.ds(j * 16, 16)] = jnp.where(valid, d4 + h, dump)
            return 0
        lax.fori_loop(0, 8, step, 0)
        for h in range(4):
            pltpu.sync_copy(abufs[h].at[pl.ds(w * WIN, WIN)],
                            den_sp.at[scws[h]], add=True)
        return 0
    lax.fori_loop(0, NWIN, window, 0)
    for h in range(4):
        pltpu.sync_copy(abufs[h].at[pl.ds(0, EW)],
                        alphaT_ref.at[pl.ds(h * E + base, EW)])
    plsc.subcore_barrier()
    pltpu.sync_copy(den_sp.at[pl.ds(s * DEN_TILE, DEN_TILE)],
                    den_ref.at[c, pl.ds(s * DEN_TILE, DEN_TILE)])


def _alpha_sc(ee_flat, src, dst):
    f = pl.kernel(
        _alpha_body,
        out_type=[
            jax.ShapeDtypeStruct((4 * E,), jnp.float32),
            jax.ShapeDtypeStruct((SC_NC, DEN_PAD), jnp.float32),
        ],
        mesh=plsc.VectorSubcoreMesh(**_SC_MESH),
        compiler_params=pltpu.CompilerParams(needs_layout_passes=False),
        scratch_types=[
            pltpu.VMEM((8 * N,), jnp.float32),
            pltpu.VMEM((EWPAD,), jnp.int32),
            pltpu.VMEM((EWPAD,), jnp.int32),
            pltpu.VMEM((EWPAD,), jnp.float32),
            pltpu.VMEM((EWPAD,), jnp.float32),
            pltpu.VMEM((EWPAD,), jnp.float32),
            pltpu.VMEM((EWPAD,), jnp.float32),
            pltpu.VMEM((WIN,), jnp.int32),
            pltpu.VMEM((WIN,), jnp.int32),
            pltpu.VMEM((WIN,), jnp.int32),
            pltpu.VMEM((WIN,), jnp.int32),
            pltpu.VMEM((DEN_TILE,), jnp.float32),
            pltpu.VMEM_SHARED((DEN_PAD,), jnp.float32),
        ],
    )
    return f(ee_flat, src, dst)


MWIN = 96
MNWIN = 54
MEPAD = MNWIN * MWIN


def _mlp_body(uvt_ref, src_ref, dst_ref, wm2_ref, b2_ref, out_ref,
              sbuf, dbuf, scob, bu0, bv0, bu1, bv1, wmb, b2b, sm0, sm1):
    c = lax.axis_index("c")
    s = lax.axis_index("s")
    wid = s * SC_NC + c
    base = wid * EW
    pltpu.sync_copy(src_ref.at[pl.ds(base, EW)], sbuf.at[pl.ds(0, EW)])
    pltpu.sync_copy(dst_ref.at[pl.ds(base, EW)], dbuf.at[pl.ds(0, EW)])
    pltpu.sync_copy(wm2_ref, wmb)
    pltpu.sync_copy(b2_ref, b2b)

    def san(i, _):
        sbuf[pl.ds(i * 16, 16)] = jnp.clip(sbuf[pl.ds(i * 16, 16)], 0, N - 1)
        dbuf[pl.ds(i * 16, 16)] = jnp.clip(dbuf[pl.ds(i * 16, 16)], 0, N - 1) + N
        return 0
    lax.fori_loop(0, MEPAD // 16, san, 0)
    b2s = b2b[pl.ds(0, 16)][0]
    iota = lax.iota(jnp.int32, 16)

    def start(w, bu, bv, sm):
        pltpu.async_copy(uvt_ref.at[sbuf.at[pl.ds(w * MWIN, MWIN)]], bu, sm)
        pltpu.async_copy(uvt_ref.at[dbuf.at[pl.ds(w * MWIN, MWIN)]], bv, sm)

    def wait(w, bu, bv, sm):
        pltpu.make_async_copy(uvt_ref.at[sbuf.at[pl.ds(w * MWIN, MWIN)]], bu, sm).wait()
        pltpu.make_async_copy(uvt_ref.at[dbuf.at[pl.ds(w * MWIN, MWIN)]], bv, sm).wait()

    def compute(w, bu, bv):
        def group(g, _):
            scv = jnp.zeros((16,), jnp.float32)
            for k in range(16):
                e = g * 16 + k
                acc = jnp.zeros((16,), jnp.float32)
                for v in range(16):
                    t = bu[e, pl.ds(v * 16, 16)] + bv[e, pl.ds(v * 16, 16)]
                    t = jnp.maximum(t, 0.0)
                    acc = acc + t * wmb[pl.ds(v * 16, 16)]
                scv = jnp.where(iota == k, jnp.sum(acc) + b2s, scv)
            scob[pl.ds(w * MWIN + g * 16, 16)] = scv
            return 0
        lax.fori_loop(0, MWIN // 16, group, 0)

    start(0, bu0, bv0, sm0)

    def pair(p, _):
        w0 = p * 2
        w1 = w0 + 1
        start(w1, bu1, bv1, sm1)
        wait(w0, bu0, bv0, sm0)
        compute(w0, bu0, bv0)

        @pl.when(p < MNWIN // 2 - 1)
        def _():
            start(w0 + 2, bu0, bv0, sm0)
        wait(w1, bu1, bv1, sm1)
        compute(w1, bu1, bv1)
        return 0
    lax.fori_loop(0, MNWIN // 2, pair, 0)
    pltpu.sync_copy(scob.at[pl.ds(0, EW)], out_ref.at[pl.ds(base, EW)])


def _mlp_sc(uvt, srccat, dstcat, wm2, b2):
    f = pl.kernel(
        _mlp_body,
        out_type=jax.ShapeDtypeStruct((E,), jnp.float32),
        mesh=plsc.VectorSubcoreMesh(**_SC_MESH),
        compiler_params=pltpu.CompilerParams(needs_layout_passes=False),
        scratch_types=[
            pltpu.VMEM((MEPAD,), jnp.int32),
            pltpu.VMEM((MEPAD,), jnp.int32),
            pltpu.VMEM((MEPAD,), jnp.float32),
            pltpu.VMEM((MWIN, 256), jnp.float32),
            pltpu.VMEM((MWIN, 256), jnp.float32),
            pltpu.VMEM((MWIN, 256), jnp.float32),
            pltpu.VMEM((MWIN, 256), jnp.float32),
            pltpu.VMEM((256,), jnp.float32),
            pltpu.VMEM((8,), jnp.float32),
            pltpu.SemaphoreType.DMA,
            pltpu.SemaphoreType.DMA,
        ],
    )
    return f(uvt, srccat, dstcat, wm2, b2)


ET = E // SC_NS
APASS = 5
PE = ET // APASS
PNWIN = 16
PEPAD = PNWIN * WIN
NPADR = 10752
ZROWS = 128
FL_A = 632
FL_B = N - 15 * FL_A


def _agg_body_gen(C, D):
    CPC = C // SC_NC

    def body(ht_ref, alphaT_ref, src_ref, dst_ref, outf_ref,
             sbuf, dbuf, alA, alB, gw0, gw1, sc0, sc1,
             rg0, rg1, rf, sm0, sm1, acc_sp):
        core = lax.axis_index("c")
        s = lax.axis_index("s")
        ebase = s * ET
        iota = lax.iota(jnp.int32, 16)
        zrow0 = s * (NPADR // SC_NS)

        def build_idx(w, c, gw, sc):
            def step(j, _):
                off = w * WIN + j * 16
                eidx = off + iota
                valid = eidx < PE
                s16 = jnp.clip(sbuf[pl.ds(off, 16)], 0, N - 1) + c * N
                d16 = jnp.clip(dbuf[pl.ds(off, 16)], 0, N - 1)
                gw[pl.ds(j * 16, 16)] = s16
                sc[pl.ds(j * 16, 16)] = jnp.where(valid, d16, N + s * 16 + iota)
                return 0
            lax.fori_loop(0, 8, step, 0)

        def scale(w, rg):
            def group(g, _):
                a16 = alA[pl.ds(w * WIN + g * 16, 16)]
                b16 = alB[pl.ds(w * WIN + g * 16, 16)]
                for k in range(16):
                    e = g * 16 + k
                    av = a16[k]
                    bv = b16[k]
                    for v in range(4):
                        x32 = rg[e, pl.ds(v * 16, 16)]
                        xbf = plsc.bitcast(x32, jnp.bfloat16)
                        ev, od = plsc.unpack(xbf, format=plsc.PackFormat.INTERLEAVED)
                        a = av if v < 2 else bv
                        rf[e, pl.ds(v * 32, 16)] = ev * a
                        rf[e, pl.ds(v * 32 + 16, 16)] = od * a
                return 0
            lax.fori_loop(0, WIN // 16, group, 0)

        def chunk_body(cc, _):
            c = cc * SC_NC + core
            ha = (c * 128) // D
            hb = (c * 128 + 64) // D

            def zfill(r, _):
                for v in range(8):
                    rf[r, pl.ds(v * 16, 16)] = jnp.zeros((16,), jnp.float32)
                return 0
            lax.fori_loop(0, ZROWS, zfill, 0)
            for z in range(5):
                pltpu.sync_copy(rf, acc_sp.at[pl.ds(zrow0 + z * ZROWS, ZROWS)])
            pltpu.sync_copy(rf.at[pl.ds(0, 32)],
                            acc_sp.at[pl.ds(zrow0 + 5 * ZROWS, 32)])
            plsc.subcore_barrier()

            def pass_body(hp, _):
                eoff = ebase + hp * PE
                pltpu.sync_copy(src_ref.at[pl.ds(eoff, PE)], sbuf.at[pl.ds(0, PE)])
                pltpu.sync_copy(dst_ref.at[pl.ds(eoff, PE)], dbuf.at[pl.ds(0, PE)])
                pltpu.sync_copy(alphaT_ref.at[pl.ds(ha * E + eoff, PE)],
                                alA.at[pl.ds(0, PE)])
                pltpu.sync_copy(alphaT_ref.at[pl.ds(hb * E + eoff, PE)],
                                alB.at[pl.ds(0, PE)])

                build_idx(0, c, gw0, sc0)
                pltpu.async_copy(ht_ref.at[gw0], rg0, sm0)

                def pair(p, _):
                    w0 = p * 2
                    w1 = w0 + 1
                    build_idx(w1, c, gw1, sc1)
                    pltpu.async_copy(ht_ref.at[gw1], rg1, sm1)
                    pltpu.make_async_copy(ht_ref.at[gw0], rg0, sm0).wait()
                    scale(w0, rg0)
                    pltpu.sync_copy(rf, acc_sp.at[sc0], add=True)

                    @pl.when(p < PNWIN // 2 - 1)
                    def _():
                        build_idx(w0 + 2, c, gw0, sc0)
                        pltpu.async_copy(ht_ref.at[gw0], rg0, sm0)
                    pltpu.make_async_copy(ht_ref.at[gw1], rg1, sm1).wait()
                    scale(w1, rg1)
                    pltpu.sync_copy(rf, acc_sp.at[sc1], add=True)
                    return 0
                lax.fori_loop(0, PNWIN // 2, pair, 0)
                return 0
            lax.fori_loop(0, APASS, pass_body, 0)
            plsc.subcore_barrier()

            @pl.when(s < SC_NS - 1)
            def _():
                pltpu.sync_copy(acc_sp.at[pl.ds(s * FL_A, FL_A)],
                                outf_ref.at[pl.ds(c * N + s * FL_A, FL_A)])

            @pl.when(s == SC_NS - 1)
            def _():
                pltpu.sync_copy(acc_sp.at[pl.ds(15 * FL_A, FL_B)],
                                outf_ref.at[pl.ds(c * N + 15 * FL_A, FL_B)])
            plsc.subcore_barrier()
            return 0
        lax.fori_loop(0, CPC, chunk_body, 0)
    return body


def _agg_sc(ht_bf, alphaT, src, dst, C, D):
    ht_flat = jax.lax.bitcast_convert_type(
        ht_bf.reshape(C * N, 64, 2), jnp.int32)
    f = pl.kernel(
        _agg_body_gen(C, D),
        out_type=jax.ShapeDtypeStruct((C * N, 128), jnp.float32),
        mesh=plsc.VectorSubcoreMesh(**_SC_MESH),
        compiler_params=pltpu.CompilerParams(needs_layout_passes=False,
                                             use_tc_tiling_on_sc=False),
        scratch_types=[
            pltpu.VMEM((PEPAD,), jnp.int32),
            pltpu.VMEM((PEPAD,), jnp.int32),
            pltpu.VMEM((PEPAD,), jnp.float32),
            pltpu.VMEM((PEPAD,), jnp.float32),
            pltpu.VMEM((WIN,), jnp.int32),
            pltpu.VMEM((WIN,), jnp.int32),
            pltpu.VMEM((WIN,), jnp.int32),
            pltpu.VMEM((WIN,), jnp.int32),
            pltpu.VMEM((WIN, 64), jnp.int32),
            pltpu.VMEM((WIN, 64), jnp.int32),
            pltpu.VMEM((WIN, 128), jnp.float32),
            pltpu.SemaphoreType.DMA,
            pltpu.SemaphoreType.DMA,
            pltpu.VMEM_SHARED((NPADR, 128), jnp.float32),
        ],
    )
    return f(ht_flat, alphaT, src, dst)


def _edge_phase(ht, ee, src, dst, C, D):
    alphaT, den2x = _alpha_sc(ee.reshape(8 * N), src, dst)
    outf = _agg_sc(ht.reshape(C * N, 128), alphaT, src, dst, C, D)
    return outf.reshape(C, N, 128), den2x[:, :4 * N].reshape(2, N, 4)


def kernel(x, block_edge_index, pos_edge_index, neg_edge_index,
           Wg1, al1, ar1, Wg2, al2, ar2, Wm1, bm1, Wm2, bm2):
    eye4 = jnp.eye(4, dtype=jnp.float32)
    AA1 = jnp.concatenate(
        [(eye4[:, None, :] * al1[:, :, None]).reshape(1024, 4),
         (eye4[:, None, :] * ar1[:, :, None]).reshape(1024, 4)], axis=1)
    AA2 = jnp.concatenate(
        [(eye4[:, None, :] * al2[:, :, None]).reshape(256, 4),
         (eye4[:, None, :] * ar2[:, :, None]).reshape(256, 4)], axis=1)

    def _perm(F):
        blk = jnp.arange(32).reshape(2, 16).T.reshape(32)
        return (jnp.arange(0, F, 32)[:, None] + blk[None, :]).reshape(F)

    P1 = _perm(1024)
    P2 = _perm(256)
    Wg1p = Wg1[:, P1]
    AA1p = AA1[P1]
    Wg2rp = Wg2[:, P2].reshape(8, 128, 256)
    AA2p = AA2[P2]
    src, dst = block_edge_index[0], block_edge_index[1]

    h1t, ee1 = _mm1(x, Wg1p, AA1p, 8, 512)
    out1, den1 = _edge_phase(h1t, ee1, src, dst, 8, 256)
    h2t, ee2 = _mid(out1, den1, Wg2rp, AA2p)
    out2, den2 = _edge_phase(h2t, ee2, src, dst, 2, 64)
    uv = _uv(out2, den2, Wm1, bm1.reshape(1, 256))
    srccat = jnp.concatenate([pos_edge_index[0], neg_edge_index[0]])
    dstcat = jnp.concatenate([pos_edge_index[1], neg_edge_index[1]])
    scores = _mlp_sc(uv.reshape(2 * N, 256), srccat, dstcat,
                     Wm2.reshape(256), jnp.broadcast_to(bm2, (8,)))
    return (scores[:EP], scores[EP:])


# speedup vs baseline: 1.6373x; 1.5233x over previous
import functools

import jax
import jax.numpy as jnp
from jax import lax
from jax.experimental import pallas as pl
from jax.experimental.pallas import tpu as pltpu
from jax.experimental.pallas import tpu_sc as plsc

N = 10000
E = 160000
EP = 80000
RB = 1000
NRB = N // RB


def _mm1_body(x_ref, w_ref, aa_ref, h_ref, ee_ref):
    c = pl.program_id(1)
    hb = jnp.dot(x_ref[...], w_ref[...], preferred_element_type=jnp.float32)
    h_ref[0] = hb

    @pl.when(c == 0)
    def _():
        ee_ref[...] = jnp.zeros_like(ee_ref)

    ee_ref[...] += jnp.dot(hb, aa_ref[...], preferred_element_type=jnp.float32)


def _mm1(x, Wg1, AA1, C, K):
    return pl.pallas_call(
        _mm1_body,
        grid=(NRB, C),
        in_specs=[
            pl.BlockSpec((RB, K), lambda r, c: (r, 0)),
            pl.BlockSpec((K, 128), lambda r, c: (0, c)),
            pl.BlockSpec((128, 8), lambda r, c: (c, 0)),
        ],
        out_specs=[
            pl.BlockSpec((1, RB, 128), lambda r, c: (c, r, 0)),
            pl.BlockSpec((RB, 8), lambda r, c: (r, 0)),
        ],
        out_shape=[
            jax.ShapeDtypeStruct((C, N, 128), jnp.float32),
            jax.ShapeDtypeStruct((N, 8), jnp.float32),
        ],
    )(x, Wg1, AA1)


def _mid_body(out1_ref, den_ref, wg2_ref, aa2_ref, h2t_ref, ee2_ref):
    den = jnp.maximum(den_ref[0] + den_ref[1], 1e-9)
    acc = jnp.zeros((RB, 256), jnp.float32)
    for c in range(8):
        xc = out1_ref[c] / den[:, c // 2][:, None]
        xc = jnp.where(xc > 0, xc, jnp.exp(jnp.minimum(xc, 0.0)) - 1.0)
        acc += jnp.dot(xc, wg2_ref[c], preferred_element_type=jnp.float32)
    h2t_ref[0] = acc[:, :128]
    h2t_ref[1] = acc[:, 128:]
    ee2_ref[...] = jnp.dot(acc, aa2_ref[...], preferred_element_type=jnp.float32)


def _mid(out1, den2x, Wg2r, AA2):
    return pl.pallas_call(
        _mid_body,
        grid=(NRB,),
        in_specs=[
            pl.BlockSpec((8, RB, 128), lambda r: (0, r, 0)),
            pl.BlockSpec((2, RB, 4), lambda r: (0, r, 0)),
            pl.BlockSpec((8, 128, 256), lambda r: (0, 0, 0)),
            pl.BlockSpec((256, 8), lambda r: (0, 0)),
        ],
        out_specs=[
            pl.BlockSpec((2, RB, 128), lambda r: (0, r, 0)),
            pl.BlockSpec((RB, 8), lambda r: (r, 0)),
        ],
        out_shape=[
            jax.ShapeDtypeStruct((2, N, 128), jnp.float32),
            jax.ShapeDtypeStruct((N, 8), jnp.float32),
        ],
    )(out1, den2x, Wg2r, AA2)


def _uv_body(out2_ref, den_ref, wm1_ref, b1_ref, uv_ref):
    den = jnp.maximum(den_ref[0] + den_ref[1], 1e-9)
    xs = []
    for c in range(2):
        div = jnp.concatenate(
            [jnp.broadcast_to(den[:, 2 * c][:, None], (RB, 64)),
             jnp.broadcast_to(den[:, 2 * c + 1][:, None], (RB, 64))], axis=1)
        xs.append(out2_ref[c] / div)
    u = (jnp.dot(xs[0], wm1_ref[0:128], preferred_element_type=jnp.float32)
         + jnp.dot(xs[1], wm1_ref[128:256], preferred_element_type=jnp.float32)
         + b1_ref[...])
    v = (jnp.dot(xs[0], wm1_ref[256:384], preferred_element_type=jnp.float32)
         + jnp.dot(xs[1], wm1_ref[384:512], preferred_element_type=jnp.float32))
    uv_ref[0] = u
    uv_ref[1] = v


def _uv(out2, den2x, Wm1, b1):
    return pl.pallas_call(
        _uv_body,
        grid=(NRB,),
        in_specs=[
            pl.BlockSpec((2, RB, 128), lambda r: (0, r, 0)),
            pl.BlockSpec((2, RB, 4), lambda r: (0, r, 0)),
            pl.BlockSpec((512, 256), lambda r: (0, 0)),
            pl.BlockSpec((1, 256), lambda r: (0, 0)),
        ],
        out_specs=pl.BlockSpec((2, RB, 256), lambda r: (0, r, 0)),
        out_shape=jax.ShapeDtypeStruct((2, N, 256), jnp.float32),
    )(out2, den2x, Wm1, b1)


SC_NC = 2
SC_NS = 16
NWK = SC_NC * SC_NS
EW = E // NWK
WIN = 128
NWIN = (EW + WIN - 1) // WIN
EWPAD = NWIN * WIN
DEN_PAD = 40960
DEN_TILE = DEN_PAD // SC_NS

_SC_MESH = dict(core_axis_name="c", subcore_axis_name="s")


def _alpha_body(ee_ref, src_ref, dst_ref, alphaT_ref, den_ref,
                eebuf, sbuf, dbuf, ab0, ab1, ab2, ab3,
                scw0, scw1, scw2, scw3, zwin, den_sp):
    c = lax.axis_index("c")
    s = lax.axis_index("s")
    wid = s * SC_NC + c
    base = wid * EW
    scws = (scw0, scw1, scw2, scw3)
    abufs = (ab0, ab1, ab2, ab3)
    pltpu.sync_copy(ee_ref, eebuf)
    pltpu.sync_copy(src_ref.at[pl.ds(base, EW)], sbuf.at[pl.ds(0, EW)])
    pltpu.sync_copy(dst_ref.at[pl.ds(base, EW)], dbuf.at[pl.ds(0, EW)])

    def zloop(i, _):
        zwin[pl.ds(i * 16, 16)] = jnp.zeros((16,), jnp.float32)
        return 0
    lax.fori_loop(0, DEN_TILE // 16, zloop, 0)
    pltpu.sync_copy(zwin, den_sp.at[pl.ds(s * DEN_TILE, DEN_TILE)])
    plsc.subcore_barrier()
    iota = lax.iota(jnp.int32, 16)

    def window(w, _):
        def step(j, _):
            off = w * WIN + j * 16
            eidx = off + iota
            valid = eidx < EW
            s16 = jnp.clip(sbuf[pl.ds(off, 16)], 0, N - 1)
            d16 = jnp.clip(dbuf[pl.ds(off, 16)], 0, N - 1)
            d4 = d16 * 4
            dump = 4 * N + wid * 16 + iota
            for h in range(4):
                el = plsc.load_gather(eebuf, [s16 * 8 + h])
                er = plsc.load_gather(eebuf, [d16 * 8 + 4 + h])
                e = el + er
                e = jnp.where(e >= 0, e, 0.2 * e)
                abufs[h][pl.ds(off, 16)] = jnp.exp(e)
                scws[h][pl.ds(j * 16, 16)] = jnp.where(valid, d4 + h, dump)
            return 0
        lax.fori_loop(0, 8, step, 0)
        for h in range(4):
            pltpu.sync_copy(abufs[h].at[pl.ds(w * WIN, WIN)],
                            den_sp.at[scws[h]], add=True)
        return 0
    lax.fori_loop(0, NWIN, window, 0)
    for h in range(4):
        pltpu.sync_copy(abufs[h].at[pl.ds(0, EW)],
                        alphaT_ref.at[pl.ds(h * E + base, EW)])
    plsc.subcore_barrier()
    pltpu.sync_copy(den_sp.at[pl.ds(s * DEN_TILE, DEN_TILE)],
                    den_ref.at[c, pl.ds(s * DEN_TILE, DEN_TILE)])


def _alpha_sc(ee_flat, src, dst):
    f = pl.kernel(
        _alpha_body,
        out_type=[
            jax.ShapeDtypeStruct((4 * E,), jnp.float32),
            jax.ShapeDtypeStruct((SC_NC, DEN_PAD), jnp.float32),
        ],
        mesh=plsc.VectorSubcoreMesh(**_SC_MESH),
        compiler_params=pltpu.CompilerParams(needs_layout_passes=False),
        scratch_types=[
            pltpu.VMEM((8 * N,), jnp.float32),
            pltpu.VMEM((EWPAD,), jnp.int32),
            pltpu.VMEM((EWPAD,), jnp.int32),
            pltpu.VMEM((EWPAD,), jnp.float32),
            pltpu.VMEM((EWPAD,), jnp.float32),
            pltpu.VMEM((EWPAD,), jnp.float32),
            pltpu.VMEM((EWPAD,), jnp.float32),
            pltpu.VMEM((WIN,), jnp.int32),
            pltpu.VMEM((WIN,), jnp.int32),
            pltpu.VMEM((WIN,), jnp.int32),
            pltpu.VMEM((WIN,), jnp.int32),
            pltpu.VMEM((DEN_TILE,), jnp.float32),
            pltpu.VMEM_SHARED((DEN_PAD,), jnp.float32),
        ],
    )
    return f(ee_flat, src, dst)


MWIN = 96
MNWIN = 54
MEPAD = MNWIN * MWIN


def _mlp_body(uvt_ref, src_ref, dst_ref, wm2_ref, b2_ref, out_ref,
              sbuf, dbuf, scob, bu0, bv0, bu1, bv1, wmb, b2b, sm0, sm1):
    c = lax.axis_index("c")
    s = lax.axis_index("s")
    wid = s * SC_NC + c
    base = wid * EW
    pltpu.sync_copy(src_ref.at[pl.ds(base, EW)], sbuf.at[pl.ds(0, EW)])
    pltpu.sync_copy(dst_ref.at[pl.ds(base, EW)], dbuf.at[pl.ds(0, EW)])
    pltpu.sync_copy(wm2_ref, wmb)
    pltpu.sync_copy(b2_ref, b2b)

    def san(i, _):
        sbuf[pl.ds(i * 16, 16)] = jnp.clip(sbuf[pl.ds(i * 16, 16)], 0, N - 1)
        dbuf[pl.ds(i * 16, 16)] = jnp.clip(dbuf[pl.ds(i * 16, 16)], 0, N - 1) + N
        return 0
    lax.fori_loop(0, MEPAD // 16, san, 0)
    b2s = b2b[pl.ds(0, 16)][0]
    iota = lax.iota(jnp.int32, 16)

    def start(w, bu, bv, sm):
        pltpu.async_copy(uvt_ref.at[sbuf.at[pl.ds(w * MWIN, MWIN)]], bu, sm)
        pltpu.async_copy(uvt_ref.at[dbuf.at[pl.ds(w * MWIN, MWIN)]], bv, sm)

    def wait(w, bu, bv, sm):
        pltpu.make_async_copy(uvt_ref.at[sbuf.at[pl.ds(w * MWIN, MWIN)]], bu, sm).wait()
        pltpu.make_async_copy(uvt_ref.at[dbuf.at[pl.ds(w * MWIN, MWIN)]], bv, sm).wait()

    def compute(w, bu, bv):
        def group(g, _):
            scv = jnp.zeros((16,), jnp.float32)
            for k in range(16):
                e = g * 16 + k
                acc = jnp.zeros((16,), jnp.float32)
                for v in range(16):
                    t = bu[e, pl.ds(v * 16, 16)] + bv[e, pl.ds(v * 16, 16)]
                    t = jnp.maximum(t, 0.0)
                    acc = acc + t * wmb[pl.ds(v * 16, 16)]
                scv = jnp.where(iota == k, jnp.sum(acc) + b2s, scv)
            scob[pl.ds(w * MWIN + g * 16, 16)] = scv
            return 0
        lax.fori_loop(0, MWIN // 16, group, 0)

    start(0, bu0, bv0, sm0)

    def pair(p, _):
        w0 = p * 2
        w1 = w0 + 1
        start(w1, bu1, bv1, sm1)
        wait(w0, bu0, bv0, sm0)
        compute(w0, bu0, bv0)

        @pl.when(p < MNWIN // 2 - 1)
        def _():
            start(w0 + 2, bu0, bv0, sm0)
        wait(w1, bu1, bv1, sm1)
        compute(w1, bu1, bv1)
        return 0
    lax.fori_loop(0, MNWIN // 2, pair, 0)
    pltpu.sync_copy(scob.at[pl.ds(0, EW)], out_ref.at[pl.ds(base, EW)])


def _mlp_sc(uvt, srccat, dstcat, wm2, b2):
    f = pl.kernel(
        _mlp_body,
        out_type=jax.ShapeDtypeStruct((E,), jnp.float32),
        mesh=plsc.VectorSubcoreMesh(**_SC_MESH),
        compiler_params=pltpu.CompilerParams(needs_layout_passes=False),
        scratch_types=[
            pltpu.VMEM((MEPAD,), jnp.int32),
            pltpu.VMEM((MEPAD,), jnp.int32),
            pltpu.VMEM((MEPAD,), jnp.float32),
            pltpu.VMEM((MWIN, 256), jnp.float32),
            pltpu.VMEM((MWIN, 256), jnp.float32),
            pltpu.VMEM((MWIN, 256), jnp.float32),
            pltpu.VMEM((MWIN, 256), jnp.float32),
            pltpu.VMEM((256,), jnp.float32),
            pltpu.VMEM((8,), jnp.float32),
            pltpu.SemaphoreType.DMA,
            pltpu.SemaphoreType.DMA,
        ],
    )
    return f(uvt, srccat, dstcat, wm2, b2)


ET = E // SC_NS
APASS = 5
PE = ET // APASS
PNWIN = 16
PEPAD = PNWIN * WIN
NPADR = 10752
ZROWS = 128
FL_A = 632
FL_B = N - 15 * FL_A


def _agg_body_gen(C, D):
    CPC = C // SC_NC

    def body(ht_ref, alphaT_ref, src_ref, dst_ref, outf_ref,
             sbuf, dbuf, alA, alB, gw0, gw1, sc0, sc1,
             rows0, rows1, sm0, sm1, acc_sp):
        core = lax.axis_index("c")
        s = lax.axis_index("s")
        ebase = s * ET
        iota = lax.iota(jnp.int32, 16)
        zrow0 = s * (NPADR // SC_NS)

        def build_idx(w, c, gw, sc):
            def step(j, _):
                off = w * WIN + j * 16
                eidx = off + iota
                valid = eidx < PE
                s16 = jnp.clip(sbuf[pl.ds(off, 16)], 0, N - 1) + c * N
                d16 = jnp.clip(dbuf[pl.ds(off, 16)], 0, N - 1)
                gw[pl.ds(j * 16, 16)] = s16
                sc[pl.ds(j * 16, 16)] = jnp.where(valid, d16, N + s * 16 + iota)
                return 0
            lax.fori_loop(0, 8, step, 0)

        def scale(w, rows):
            def group(g, _):
                a16 = alA[pl.ds(w * WIN + g * 16, 16)]
                b16 = alB[pl.ds(w * WIN + g * 16, 16)]
                for k in range(16):
                    e = g * 16 + k
                    av = a16[k]
                    bv = b16[k]
                    for v in range(4):
                        rows[e, pl.ds(v * 16, 16)] *= av
                    for v in range(4, 8):
                        rows[e, pl.ds(v * 16, 16)] *= bv
                return 0
            lax.fori_loop(0, WIN // 16, group, 0)

        def chunk_body(cc, _):
            c = cc * SC_NC + core
            ha = (c * 128) // D
            hb = (c * 128 + 64) // D

            def zfill(r, _):
                for v in range(8):
                    rows0[r, pl.ds(v * 16, 16)] = jnp.zeros((16,), jnp.float32)
                return 0
            lax.fori_loop(0, ZROWS, zfill, 0)
            for z in range(5):
                pltpu.sync_copy(rows0, acc_sp.at[pl.ds(zrow0 + z * ZROWS, ZROWS)])
            pltpu.sync_copy(rows0.at[pl.ds(0, 32)],
                            acc_sp.at[pl.ds(zrow0 + 5 * ZROWS, 32)])
            plsc.subcore_barrier()

            def pass_body(hp, _):
                eoff = ebase + hp * PE
                pltpu.sync_copy(src_ref.at[pl.ds(eoff, PE)], sbuf.at[pl.ds(0, PE)])
                pltpu.sync_copy(dst_ref.at[pl.ds(eoff, PE)], dbuf.at[pl.ds(0, PE)])
                pltpu.sync_copy(alphaT_ref.at[pl.ds(ha * E + eoff, PE)],
                                alA.at[pl.ds(0, PE)])
                pltpu.sync_copy(alphaT_ref.at[pl.ds(hb * E + eoff, PE)],
                                alB.at[pl.ds(0, PE)])

                build_idx(0, c, gw0, sc0)
                pltpu.async_copy(ht_ref.at[gw0], rows0, sm0)

                def pair(p, _):
                    w0 = p * 2
                    w1 = w0 + 1
                    build_idx(w1, c, gw1, sc1)
                    pltpu.async_copy(ht_ref.at[gw1], rows1, sm1)
                    pltpu.make_async_copy(ht_ref.at[gw0], rows0, sm0).wait()
                    scale(w0, rows0)
                    pltpu.sync_copy(rows0, acc_sp.at[sc0], add=True)

                    @pl.when(p < PNWIN // 2 - 1)
                    def _():
                        build_idx(w0 + 2, c, gw0, sc0)
                        pltpu.async_copy(ht_ref.at[gw0], rows0, sm0)
                    pltpu.make_async_copy(ht_ref.at[gw1], rows1, sm1).wait()
                    scale(w1, rows1)
                    pltpu.sync_copy(rows1, acc_sp.at[sc1], add=True)
                    return 0
                lax.fori_loop(0, PNWIN // 2, pair, 0)
                return 0
            lax.fori_loop(0, APASS, pass_body, 0)
            plsc.subcore_barrier()

            @pl.when(s < SC_NS - 1)
            def _():
                pltpu.sync_copy(acc_sp.at[pl.ds(s * FL_A, FL_A)],
                                outf_ref.at[pl.ds(c * N + s * FL_A, FL_A)])

            @pl.when(s == SC_NS - 1)
            def _():
                pltpu.sync_copy(acc_sp.at[pl.ds(15 * FL_A, FL_B)],
                                outf_ref.at[pl.ds(c * N + 15 * FL_A, FL_B)])
            plsc.subcore_barrier()
            return 0
        lax.fori_loop(0, CPC, chunk_body, 0)
    return body


def _agg_sc(ht_flat, alphaT, src, dst, C, D):
    f = pl.kernel(
        _agg_body_gen(C, D),
        out_type=jax.ShapeDtypeStruct((C * N, 128), jnp.float32),
        mesh=plsc.VectorSubcoreMesh(**_SC_MESH),
        compiler_params=pltpu.CompilerParams(needs_layout_passes=False),
        scratch_types=[
            pltpu.VMEM((PEPAD,), jnp.int32),
            pltpu.VMEM((PEPAD,), jnp.int32),
            pltpu.VMEM((PEPAD,), jnp.float32),
            pltpu.VMEM((PEPAD,), jnp.float32),
            pltpu.VMEM((WIN,), jnp.int32),
            pltpu.VMEM((WIN,), jnp.int32),
            pltpu.VMEM((WIN,), jnp.int32),
            pltpu.VMEM((WIN,), jnp.int32),
            pltpu.VMEM((WIN, 128), jnp.float32),
            pltpu.VMEM((WIN, 128), jnp.float32),
            pltpu.SemaphoreType.DMA,
            pltpu.SemaphoreType.DMA,
            pltpu.VMEM_SHARED((NPADR, 128), jnp.float32),
        ],
    )
    return f(ht_flat, alphaT, src, dst)


def _edge_phase(ht, ee, src, dst, C, D):
    alphaT, den2x = _alpha_sc(ee.reshape(8 * N), src, dst)
    outf = _agg_sc(ht.reshape(C * N, 128), alphaT, src, dst, C, D)
    return outf.reshape(C, N, 128), den2x[:, :4 * N].reshape(2, N, 4)


def kernel(x, block_edge_index, pos_edge_index, neg_edge_index,
           Wg1, al1, ar1, Wg2, al2, ar2, Wm1, bm1, Wm2, bm2):
    eye4 = jnp.eye(4, dtype=jnp.float32)
    AA1 = jnp.concatenate(
        [(eye4[:, None, :] * al1[:, :, None]).reshape(1024, 4),
         (eye4[:, None, :] * ar1[:, :, None]).reshape(1024, 4)], axis=1)
    AA2 = jnp.concatenate(
        [(eye4[:, None, :] * al2[:, :, None]).reshape(256, 4),
         (eye4[:, None, :] * ar2[:, :, None]).reshape(256, 4)], axis=1)

    Wg2r = Wg2.reshape(8, 128, 256)
    src, dst = block_edge_index[0], block_edge_index[1]

    h1t, ee1 = _mm1(x, Wg1, AA1, 8, 512)
    out1, den1 = _edge_phase(h1t, ee1, src, dst, 8, 256)
    h2t, ee2 = _mid(out1, den1, Wg2r, AA2)
    out2, den2 = _edge_phase(h2t, ee2, src, dst, 2, 64)
    uv = _uv(out2, den2, Wm1, bm1.reshape(1, 256))
    srccat = jnp.concatenate([pos_edge_index[0], neg_edge_index[0]])
    dstcat = jnp.concatenate([pos_edge_index[1], neg_edge_index[1]])
    scores = _mlp_sc(uv.reshape(2 * N, 256), srccat, dstcat,
                     Wm2.reshape(256), jnp.broadcast_to(bm2, (8,)))
    return (scores[:EP], scores[EP:])


# speedup vs baseline: 1.6403x; 1.0019x over previous
import jax
import jax.numpy as jnp
from jax import lax
from jax.experimental import pallas as pl
from jax.experimental.pallas import tpu as pltpu
from jax.experimental.pallas import tpu_sc as plsc

N = 10000
E = 160000
EP = 80000
RB = 1000
NRB = N // RB


def _mm1_body(x_ref, w_ref, aa_ref, h_ref, ee_ref):
    c = pl.program_id(1)
    hb = jnp.dot(x_ref[...], w_ref[...], preferred_element_type=jnp.float32)
    h_ref[0] = hb

    @pl.when(c == 0)
    def _():
        ee_ref[...] = jnp.zeros_like(ee_ref)

    ee_ref[...] += jnp.dot(hb, aa_ref[...], preferred_element_type=jnp.float32)


def _mm1(x, Wg1, AA1, C, K):
    return pl.pallas_call(
        _mm1_body,
        grid=(NRB, C),
        in_specs=[
            pl.BlockSpec((RB, K), lambda r, c: (r, 0)),
            pl.BlockSpec((K, 128), lambda r, c: (0, c)),
            pl.BlockSpec((128, 8), lambda r, c: (c, 0)),
        ],
        out_specs=[
            pl.BlockSpec((1, RB, 128), lambda r, c: (c, r, 0)),
            pl.BlockSpec((RB, 8), lambda r, c: (r, 0)),
        ],
        out_shape=[
            jax.ShapeDtypeStruct((C, N, 128), jnp.float32),
            jax.ShapeDtypeStruct((N, 8), jnp.float32),
        ],
    )(x, Wg1, AA1)


def _mid_body(out1_ref, den_ref, wg2_ref, aa2_ref, h2t_ref, ee2_ref):
    den = jnp.maximum(den_ref[0] + den_ref[1], 1e-9)
    acc = jnp.zeros((RB, 256), jnp.float32)
    for c in range(8):
        xc = out1_ref[c] / den[:, c // 2][:, None]
        xc = jnp.where(xc > 0, xc, jnp.exp(jnp.minimum(xc, 0.0)) - 1.0)
        acc += jnp.dot(xc, wg2_ref[c], preferred_element_type=jnp.float32)
    h2t_ref[0] = acc[:, :128]
    h2t_ref[1] = acc[:, 128:]
    ee2_ref[...] = jnp.dot(acc, aa2_ref[...], preferred_element_type=jnp.float32)


def _mid(out1, den2x, Wg2r, AA2):
    return pl.pallas_call(
        _mid_body,
        grid=(NRB,),
        in_specs=[
            pl.BlockSpec((8, RB, 128), lambda r: (0, r, 0)),
            pl.BlockSpec((2, RB, 4), lambda r: (0, r, 0)),
            pl.BlockSpec((8, 128, 256), lambda r: (0, 0, 0)),
            pl.BlockSpec((256, 8), lambda r: (0, 0)),
        ],
        out_specs=[
            pl.BlockSpec((2, RB, 128), lambda r: (0, r, 0)),
            pl.BlockSpec((RB, 8), lambda r: (r, 0)),
        ],
        out_shape=[
            jax.ShapeDtypeStruct((2, N, 128), jnp.float32),
            jax.ShapeDtypeStruct((N, 8), jnp.float32),
        ],
    )(out1, den2x, Wg2r, AA2)


def _uv_body(out2_ref, den_ref, wm1_ref, b1_ref, uv_ref):
    den = jnp.maximum(den_ref[0] + den_ref[1], 1e-9)
    xs = []
    for c in range(2):
        div = jnp.concatenate(
            [jnp.broadcast_to(den[:, 2 * c][:, None], (RB, 64)),
             jnp.broadcast_to(den[:, 2 * c + 1][:, None], (RB, 64))], axis=1)
        xs.append(out2_ref[c] / div)
    u = (jnp.dot(xs[0], wm1_ref[0:128], preferred_element_type=jnp.float32)
         + jnp.dot(xs[1], wm1_ref[128:256], preferred_element_type=jnp.float32)
         + b1_ref[...])
    v = (jnp.dot(xs[0], wm1_ref[256:384], preferred_element_type=jnp.float32)
         + jnp.dot(xs[1], wm1_ref[384:512], preferred_element_type=jnp.float32))
    uv_ref[0] = u
    uv_ref[1] = v


def _uv(out2, den2x, Wm1, b1):
    return pl.pallas_call(
        _uv_body,
        grid=(NRB,),
        in_specs=[
            pl.BlockSpec((2, RB, 128), lambda r: (0, r, 0)),
            pl.BlockSpec((2, RB, 4), lambda r: (0, r, 0)),
            pl.BlockSpec((512, 256), lambda r: (0, 0)),
            pl.BlockSpec((1, 256), lambda r: (0, 0)),
        ],
        out_specs=pl.BlockSpec((2, RB, 256), lambda r: (0, r, 0)),
        out_shape=jax.ShapeDtypeStruct((2, N, 256), jnp.float32),
    )(out2, den2x, Wm1, b1)


SC_NC = 2
SC_NS = 16
NWK = SC_NC * SC_NS
EW = E // NWK
WIN = 128
NWIN = (EW + WIN - 1) // WIN
EWPAD = NWIN * WIN
DEN_PAD = 40960
DEN_TILE = DEN_PAD // SC_NS

_SC_MESH = dict(core_axis_name="c", subcore_axis_name="s")


def _alpha_body(ee_ref, src_ref, dst_ref, alphaT_ref, den_ref,
                eebuf, sbuf, dbuf, ab0, ab1, ab2, ab3,
                scw0, scw1, scw2, scw3, zwin, den_sp):
    c = lax.axis_index("c")
    s = lax.axis_index("s")
    wid = s * SC_NC + c
    base = wid * EW
    scws = (scw0, scw1, scw2, scw3)
    abufs = (ab0, ab1, ab2, ab3)
    pltpu.sync_copy(ee_ref, eebuf)
    pltpu.sync_copy(src_ref.at[pl.ds(base, EW)], sbuf.at[pl.ds(0, EW)])
    pltpu.sync_copy(dst_ref.at[pl.ds(base, EW)], dbuf.at[pl.ds(0, EW)])

    def zloop(i, _):
        zwin[pl.ds(i * 16, 16)] = jnp.zeros((16,), jnp.float32)
        return 0
    lax.fori_loop(0, DEN_TILE // 16, zloop, 0)
    pltpu.sync_copy(zwin, den_sp.at[pl.ds(s * DEN_TILE, DEN_TILE)])
    plsc.subcore_barrier()
    iota = lax.iota(jnp.int32, 16)

    def window(w, _):
        def step(j, _):
            off = w * WIN + j * 16
            eidx = off + iota
            valid = eidx < EW
            s16 = jnp.clip(sbuf[pl.ds(off, 16)], 0, N - 1)
            d16 = jnp.clip(dbuf[pl.ds(off, 16)], 0, N - 1)
            d4 = d16 * 4
            dump = 4 * N + wid * 16 + iota
            for h in range(4):
                el = plsc.load_gather(eebuf, [s16 * 8 + h])
                er = plsc.load_gather(eebuf, [d16 * 8 + 4 + h])
                e = el + er
                e = jnp.where(e >= 0, e, 0.2 * e)
                abufs[h][pl.ds(off, 16)] = jnp.exp(e)
                scws[h][pl.ds(j * 16, 16)] = jnp.where(valid, d4 + h, dump)
            return 0
        lax.fori_loop(0, 8, step, 0)
        for h in range(4):
            pltpu.sync_copy(abufs[h].at[pl.ds(w * WIN, WIN)],
                            den_sp.at[scws[h]], add=True)
        return 0
    lax.fori_loop(0, NWIN, window, 0)
    for h in range(4):
        pltpu.sync_copy(abufs[h].at[pl.ds(0, EW)],
                        alphaT_ref.at[pl.ds(h * E + base, EW)])
    plsc.subcore_barrier()
    pltpu.sync_copy(den_sp.at[pl.ds(s * DEN_TILE, DEN_TILE)],
                    den_ref.at[c, pl.ds(s * DEN_TILE, DEN_TILE)])


def _alpha_sc(ee_flat, src, dst):
    f = pl.kernel(
        _alpha_body,
        out_type=[
            jax.ShapeDtypeStruct((4 * E,), jnp.float32),
            jax.ShapeDtypeStruct((SC_NC, DEN_PAD), jnp.float32),
        ],
        mesh=plsc.VectorSubcoreMesh(**_SC_MESH),
        compiler_params=pltpu.CompilerParams(needs_layout_passes=False),
        scratch_types=[
            pltpu.VMEM((8 * N,), jnp.float32),
            pltpu.VMEM((EWPAD,), jnp.int32),
            pltpu.VMEM((EWPAD,), jnp.int32),
            pltpu.VMEM((EWPAD,), jnp.float32),
            pltpu.VMEM((EWPAD,), jnp.float32),
            pltpu.VMEM((EWPAD,), jnp.float32),
            pltpu.VMEM((EWPAD,), jnp.float32),
            pltpu.VMEM((WIN,), jnp.int32),
            pltpu.VMEM((WIN,), jnp.int32),
            pltpu.VMEM((WIN,), jnp.int32),
            pltpu.VMEM((WIN,), jnp.int32),
            pltpu.VMEM((DEN_TILE,), jnp.float32),
            pltpu.VMEM_SHARED((DEN_PAD,), jnp.float32),
        ],
    )
    return f(ee_flat, src, dst)


MWIN = 96
MNWIN = 54
MEPAD = MNWIN * MWIN


def _mlp_body(uvt_ref, src_ref, dst_ref, wm2_ref, b2_ref, out_ref,
              sbuf, dbuf, scob, bu0, bv0, bu1, bv1, wmb, b2b, sm0, sm1):
    c = lax.axis_index("c")
    s = lax.axis_index("s")
    wid = s * SC_NC + c
    base = wid * EW
    pltpu.sync_copy(src_ref.at[pl.ds(base, EW)], sbuf.at[pl.ds(0, EW)])
    pltpu.sync_copy(dst_ref.at[pl.ds(base, EW)], dbuf.at[pl.ds(0, EW)])
    pltpu.sync_copy(wm2_ref, wmb)
    pltpu.sync_copy(b2_ref, b2b)

    def san(i, _):
        sbuf[pl.ds(i * 16, 16)] = jnp.clip(sbuf[pl.ds(i * 16, 16)], 0, N - 1)
        dbuf[pl.ds(i * 16, 16)] = jnp.clip(dbuf[pl.ds(i * 16, 16)], 0, N - 1) + N
        return 0
    lax.fori_loop(0, MEPAD // 16, san, 0)
    b2s = b2b[pl.ds(0, 16)][0]
    iota = lax.iota(jnp.int32, 16)

    def start(w, bu, bv, sm):
        pltpu.async_copy(uvt_ref.at[sbuf.at[pl.ds(w * MWIN, MWIN)]], bu, sm)
        pltpu.async_copy(uvt_ref.at[dbuf.at[pl.ds(w * MWIN, MWIN)]], bv, sm)

    def wait(w, bu, bv, sm):
        pltpu.make_async_copy(uvt_ref.at[sbuf.at[pl.ds(w * MWIN, MWIN)]], bu, sm).wait()
        pltpu.make_async_copy(uvt_ref.at[dbuf.at[pl.ds(w * MWIN, MWIN)]], bv, sm).wait()

    def compute(w, bu, bv):
        def group(g, _):
            scv = jnp.zeros((16,), jnp.float32)
            for k in range(16):
                e = g * 16 + k
                acc = jnp.zeros((16,), jnp.float32)
                for v in range(16):
                    t = bu[e, pl.ds(v * 16, 16)] + bv[e, pl.ds(v * 16, 16)]
                    t = jnp.maximum(t, 0.0)
                    acc = acc + t * wmb[pl.ds(v * 16, 16)]
                scv = jnp.where(iota == k, jnp.sum(acc) + b2s, scv)
            scob[pl.ds(w * MWIN + g * 16, 16)] = scv
            return 0
        lax.fori_loop(0, MWIN // 16, group, 0)

    start(0, bu0, bv0, sm0)

    def pair(p, _):
        w0 = p * 2
        w1 = w0 + 1
        start(w1, bu1, bv1, sm1)
        wait(w0, bu0, bv0, sm0)
        compute(w0, bu0, bv0)

        @pl.when(p < MNWIN // 2 - 1)
        def _():
            start(w0 + 2, bu0, bv0, sm0)
        wait(w1, bu1, bv1, sm1)
        compute(w1, bu1, bv1)
        return 0
    lax.fori_loop(0, MNWIN // 2, pair, 0)
    pltpu.sync_copy(scob.at[pl.ds(0, EW)], out_ref.at[pl.ds(base, EW)])


def _mlp_sc(uvt, srccat, dstcat, wm2, b2):
    f = pl.kernel(
        _mlp_body,
        out_type=jax.ShapeDtypeStruct((E,), jnp.float32),
        mesh=plsc.VectorSubcoreMesh(**_SC_MESH),
        compiler_params=pltpu.CompilerParams(needs_layout_passes=False),
        scratch_types=[
            pltpu.VMEM((MEPAD,), jnp.int32),
            pltpu.VMEM((MEPAD,), jnp.int32),
            pltpu.VMEM((MEPAD,), jnp.float32),
            pltpu.VMEM((MWIN, 256), jnp.float32),
            pltpu.VMEM((MWIN, 256), jnp.float32),
            pltpu.VMEM((MWIN, 256), jnp.float32),
            pltpu.VMEM((MWIN, 256), jnp.float32),
            pltpu.VMEM((256,), jnp.float32),
            pltpu.VMEM((8,), jnp.float32),
            pltpu.SemaphoreType.DMA,
            pltpu.SemaphoreType.DMA,
        ],
    )
    return f(uvt, srccat, dstcat, wm2, b2)


ET = E // SC_NS
APASS = 5
PE = ET // APASS
PNWIN = 16
PEPAD = PNWIN * WIN
NPADR = 10752
ZROWS = 128
FL_A = 632
FL_B = N - 15 * FL_A


def _agg_body_gen(C, D):
    CPC = C // SC_NC

    def body(ht_ref, alphaT_ref, src_ref, dst_ref, outf_ref,
             sbuf, dbuf, alA, alB, gw0, gw1, sc0, sc1,
             rows0, rows1, sm0, sm1, acc_sp):
        core = lax.axis_index("c")
        s = lax.axis_index("s")
        ebase = s * ET
        iota = lax.iota(jnp.int32, 16)
        zrow0 = s * (NPADR // SC_NS)

        def build_idx(w, c, gw, sc):
            def step(j, _):
                off = w * WIN + j * 16
                eidx = off + iota
                valid = eidx < PE
                s16 = jnp.clip(sbuf[pl.ds(off, 16)], 0, N - 1) + c * N
                d16 = jnp.clip(dbuf[pl.ds(off, 16)], 0, N - 1)
                gw[pl.ds(j * 16, 16)] = s16
                sc[pl.ds(j * 16, 16)] = jnp.where(valid, d16, N + s * 16 + iota)
                return 0
            lax.fori_loop(0, 8, step, 0)

        def scale(w, rows):
            def group(g, _):
                a16 = alA[pl.ds(w * WIN + g * 16, 16)]
                b16 = alB[pl.ds(w * WIN + g * 16, 16)]
                for k in range(16):
                    e = g * 16 + k
                    av = a16[k]
                    bv = b16[k]
                    for v in range(4):
                        rows[e, pl.ds(v * 16, 16)] *= av
                    for v in range(4, 8):
                        rows[e, pl.ds(v * 16, 16)] *= bv
                return 0
            lax.fori_loop(0, WIN // 16, group, 0)

        def chunk_body(cc, _):
            c = cc * SC_NC + core
            ha = (c * 128) // D
            hb = (c * 128 + 64) // D

            def zfill(r, _):
                for v in range(8):
                    rows0[r, pl.ds(v * 16, 16)] = jnp.zeros((16,), jnp.float32)
                return 0
            lax.fori_loop(0, ZROWS, zfill, 0)
            for z in range(5):
                pltpu.sync_copy(rows0, acc_sp.at[pl.ds(zrow0 + z * ZROWS, ZROWS)])
            pltpu.sync_copy(rows0.at[pl.ds(0, 32)],
                            acc_sp.at[pl.ds(zrow0 + 5 * ZROWS, 32)])
            plsc.subcore_barrier()

            def pass_body(hp, _):
                eoff = ebase + hp * PE
                pltpu.sync_copy(src_ref.at[pl.ds(eoff, PE)], sbuf.at[pl.ds(0, PE)])
                pltpu.sync_copy(dst_ref.at[pl.ds(eoff, PE)], dbuf.at[pl.ds(0, PE)])
                pltpu.sync_copy(alphaT_ref.at[pl.ds(ha * E + eoff, PE)],
                                alA.at[pl.ds(0, PE)])
                pltpu.sync_copy(alphaT_ref.at[pl.ds(hb * E + eoff, PE)],
                                alB.at[pl.ds(0, PE)])

                build_idx(0, c, gw0, sc0)
                pltpu.async_copy(ht_ref.at[gw0], rows0, sm0)

                def pair(p, _):
                    w0 = p * 2
                    w1 = w0 + 1
                    build_idx(w1, c, gw1, sc1)
                    pltpu.async_copy(ht_ref.at[gw1], rows1, sm1)
                    pltpu.make_async_copy(ht_ref.at[gw0], rows0, sm0).wait()
                    scale(w0, rows0)
                    pltpu.sync_copy(rows0, acc_sp.at[sc0], add=True)

                    @pl.when(p < PNWIN // 2 - 1)
                    def _():
                        build_idx(w0 + 2, c, gw0, sc0)
                        pltpu.async_copy(ht_ref.at[gw0], rows0, sm0)
                    pltpu.make_async_copy(ht_ref.at[gw1], rows1, sm1).wait()
                    scale(w1, rows1)
                    pltpu.sync_copy(rows1, acc_sp.at[sc1], add=True)
                    return 0
                lax.fori_loop(0, PNWIN // 2, pair, 0)
                return 0
            lax.fori_loop(0, APASS, pass_body, 0)
            plsc.subcore_barrier()

            @pl.when(s < SC_NS - 1)
            def _():
                pltpu.sync_copy(acc_sp.at[pl.ds(s * FL_A, FL_A)],
                                outf_ref.at[pl.ds(c * N + s * FL_A, FL_A)])

            @pl.when(s == SC_NS - 1)
            def _():
                pltpu.sync_copy(acc_sp.at[pl.ds(15 * FL_A, FL_B)],
                                outf_ref.at[pl.ds(c * N + 15 * FL_A, FL_B)])
            plsc.subcore_barrier()
            return 0
        lax.fori_loop(0, CPC, chunk_body, 0)
    return body


def _agg_sc(ht_flat, alphaT, src, dst, C, D):
    f = pl.kernel(
        _agg_body_gen(C, D),
        out_type=jax.ShapeDtypeStruct((C * N, 128), jnp.float32),
        mesh=plsc.VectorSubcoreMesh(**_SC_MESH),
        compiler_params=pltpu.CompilerParams(needs_layout_passes=False),
        scratch_types=[
            pltpu.VMEM((PEPAD,), jnp.int32),
            pltpu.VMEM((PEPAD,), jnp.int32),
            pltpu.VMEM((PEPAD,), jnp.float32),
            pltpu.VMEM((PEPAD,), jnp.float32),
            pltpu.VMEM((WIN,), jnp.int32),
            pltpu.VMEM((WIN,), jnp.int32),
            pltpu.VMEM((WIN,), jnp.int32),
            pltpu.VMEM((WIN,), jnp.int32),
            pltpu.VMEM((WIN, 128), jnp.float32),
            pltpu.VMEM((WIN, 128), jnp.float32),
            pltpu.SemaphoreType.DMA,
            pltpu.SemaphoreType.DMA,
            pltpu.VMEM_SHARED((NPADR, 128), jnp.float32),
        ],
    )
    return f(ht_flat, alphaT, src, dst)


def _edge_phase(ht, ee, src, dst, C, D):
    alphaT, den2x = _alpha_sc(ee.reshape(8 * N), src, dst)
    outf = _agg_sc(ht.reshape(C * N, 128), alphaT, src, dst, C, D)
    return outf.reshape(C, N, 128), den2x[:, :4 * N].reshape(2, N, 4)


def kernel(x, block_edge_index, pos_edge_index, neg_edge_index,
           Wg1, al1, ar1, Wg2, al2, ar2, Wm1, bm1, Wm2, bm2):
    eye4 = jnp.eye(4, dtype=jnp.float32)
    AA1 = jnp.concatenate(
        [(eye4[:, None, :] * al1[:, :, None]).reshape(1024, 4),
         (eye4[:, None, :] * ar1[:, :, None]).reshape(1024, 4)], axis=1)
    AA2 = jnp.concatenate(
        [(eye4[:, None, :] * al2[:, :, None]).reshape(256, 4),
         (eye4[:, None, :] * ar2[:, :, None]).reshape(256, 4)], axis=1)

    Wg2r = Wg2.reshape(8, 128, 256)
    src, dst = block_edge_index[0], block_edge_index[1]

    h1t, ee1 = _mm1(x, Wg1, AA1, 8, 512)
    out1, den1 = _edge_phase(h1t, ee1, src, dst, 8, 256)
    h2t, ee2 = _mid(out1, den1, Wg2r, AA2)
    out2, den2 = _edge_phase(h2t, ee2, src, dst, 2, 64)
    uv = _uv(out2, den2, Wm1, bm1.reshape(1, 256))
    srccat = jnp.concatenate([pos_edge_index[0], neg_edge_index[0]])
    dstcat = jnp.concatenate([pos_edge_index[1], neg_edge_index[1]])
    scores = _mlp_sc(uv.reshape(2 * N, 256), srccat, dstcat,
                     Wm2.reshape(256), jnp.broadcast_to(bm2, (8,)))
    return (scores[:EP], scores[EP:])
